# speedup vs baseline: 11.9738x; 11.9738x over previous
import jax
import jax.numpy as jnp
from jax import lax
from jax.experimental import pallas as pl
from jax.experimental.pallas import tpu as pltpu
from jax.experimental.pallas import tpu_sc as plsc

N = 10000
NPAD = 10240
E = 320000
DI = 128
DH = 64
NCLS = 16

NC = 2
NS = 16
NW = NC * NS
EW = E // NW
CNK = 128
NCHUNK = (EW + CNK - 1) // CNK
EWPAD = NCHUNK * CNK
DUMP = 10016
RPT = NPAD // NS

_MESH = plsc.VectorSubcoreMesh(core_axis_name="c", subcore_axis_name="s")


def _hist_body(sp_hbm, dp_hbm, out_hbm, hs_sh, hd_sh, ones_v, zb_v, idx_v):
    c = lax.axis_index("c")
    s = lax.axis_index("s")
    w = c * NS + s
    for i in range(CNK // 16):
        ones_v[pl.ds(16 * i, 16)] = jnp.ones((16,), jnp.float32)
    for i in range(RPT // 16):
        zb_v[pl.ds(16 * i, 16)] = jnp.zeros((16,), jnp.float32)
    pltpu.sync_copy(zb_v, hs_sh.at[pl.ds(RPT * s, RPT)])
    pltpu.sync_copy(zb_v, hd_sh.at[pl.ds(RPT * s, RPT)])
    pltpu.sync_copy(sp_hbm.at[w], idx_v.at[0])
    pltpu.sync_copy(dp_hbm.at[w], idx_v.at[1])
    plsc.subcore_barrier()

    def step(j, carry):
        pltpu.sync_copy(ones_v, hs_sh.at[idx_v.at[0, j]], add=True)
        pltpu.sync_copy(ones_v, hd_sh.at[idx_v.at[1, j]], add=True)
        return carry

    lax.fori_loop(0, NCHUNK, step, 0)
    plsc.subcore_barrier()
    pltpu.sync_copy(hs_sh.at[pl.ds(RPT * s, RPT)],
                    out_hbm.at[c, 0, pl.ds(RPT * s, RPT)])
    pltpu.sync_copy(hd_sh.at[pl.ds(RPT * s, RPT)],
                    out_hbm.at[c, 1, pl.ds(RPT * s, RPT)])


_hist_kernel = pl.kernel(
    _hist_body,
    out_type=jax.ShapeDtypeStruct((NC, 2, NPAD), jnp.float32),
    mesh=_MESH,
    scratch_types=[
        pltpu.VMEM_SHARED((NPAD,), jnp.float32),
        pltpu.VMEM_SHARED((NPAD,), jnp.float32),
        pltpu.VMEM((CNK,), jnp.float32),
        pltpu.VMEM((RPT,), jnp.float32),
        pltpu.VMEM((2, NCHUNK, CNK), jnp.int32),
    ],
)


def _seg_body(h_hbm, sp_hbm, dp_hbm, z_hbm, out_hbm,
              h_sh, agg_sh, idx_v, rows_v):
    c = lax.axis_index("c")
    s = lax.axis_index("s")
    w = c * NS + s
    r0 = RPT * s
    pltpu.sync_copy(h_hbm.at[pl.ds(r0, RPT)], h_sh.at[pl.ds(r0, RPT)])
    pltpu.sync_copy(z_hbm, agg_sh.at[pl.ds(r0, RPT)])
    pltpu.sync_copy(sp_hbm.at[w], idx_v.at[0])
    pltpu.sync_copy(dp_hbm.at[w], idx_v.at[1])
    plsc.subcore_barrier()

    def step(j, carry):
        pltpu.sync_copy(h_sh.at[idx_v.at[0, j]], rows_v)
        pltpu.sync_copy(rows_v, agg_sh.at[idx_v.at[1, j]], add=True)
        return carry

    lax.fori_loop(0, NCHUNK, step, 0)
    plsc.subcore_barrier()
    pltpu.sync_copy(agg_sh.at[pl.ds(r0, RPT)], out_hbm.at[c, pl.ds(r0, RPT)])


_seg_kernel = pl.kernel(
    _seg_body,
    out_type=jax.ShapeDtypeStruct((NC, NPAD, DH), jnp.float32),
    mesh=_MESH,
    scratch_types=[
        pltpu.VMEM_SHARED((NPAD, DH), jnp.float32),
        pltpu.VMEM_SHARED((NPAD, DH), jnp.float32),
        pltpu.VMEM((2, NCHUNK, CNK), jnp.int32),
        pltpu.VMEM((CNK, DH), jnp.float32),
    ],
)


def _mm_body(x_ref, w_ref, deg_ref, h_ref):
    norm = lax.rsqrt(deg_ref[0] + deg_ref[1] + 1.0)
    h = jnp.dot(x_ref[...], w_ref[...], preferred_element_type=jnp.float32)
    h_ref[...] = h * norm


def _ep_body(aggp_ref, h_ref, degd_ref, b_ref, wc_ref, bc_ref, out_ref):
    agg = aggp_ref[0] + aggp_ref[1] + h_ref[...]
    norm = lax.rsqrt(degd_ref[0] + degd_ref[1] + 1.0)
    act = jnp.maximum(agg * norm + b_ref[...], 0.0)
    rid = lax.broadcasted_iota(jnp.int32, (NPAD, 1), 0)
    act = jnp.where(rid < N, act, -jnp.inf)
    hg = jnp.max(act, axis=0, keepdims=True)
    logits = lax.dot_general(hg, wc_ref[...],
                             (((1,), (1,)), ((), ()))) + bc_ref[...]
    m = jnp.max(logits, axis=1, keepdims=True)
    lse = jnp.log(jnp.sum(jnp.exp(logits - m), axis=1, keepdims=True)) + m
    out_ref[...] = logits - lse


def kernel(x, edge_index, W, b, Wc, bc):
    src = edge_index[0].reshape(NW, EW)
    dst = edge_index[1].reshape(NW, EW)
    pad = ((0, 0), (0, EWPAD - EW))
    sp = jnp.pad(src, pad, constant_values=DUMP).reshape(NW, NCHUNK, CNK)
    dp = jnp.pad(dst, pad, constant_values=DUMP).reshape(NW, NCHUNK, CNK)

    deg = _hist_kernel(sp, dp)

    x_pad = jnp.pad(x, ((0, NPAD - N), (0, 0)))
    deg_src = deg[:, 0, :].reshape(NC, NPAD, 1)
    h = pl.pallas_call(
        _mm_body,
        out_shape=jax.ShapeDtypeStruct((NPAD, DH), jnp.float32),
    )(x_pad, W, deg_src)

    zrows = jnp.zeros((RPT, DH), jnp.float32)
    aggp = _seg_kernel(h, sp, dp, zrows)

    deg_dst = deg[:, 1, :].reshape(NC, NPAD, 1)
    out = pl.pallas_call(
        _ep_body,
        out_shape=jax.ShapeDtypeStruct((1, NCLS), jnp.float32),
    )(aggp, h, deg_dst, b.reshape(1, DH), Wc, bc.reshape(1, NCLS))
    return out
